# baseline (speedup 1.0000x reference)
import functools

import jax
import jax.numpy as jnp
from jax import lax
from jax.experimental import pallas as pl
from jax.experimental.pallas import tpu as pltpu
from jax.experimental.pallas import tpu_sc as plsc

N_NODES = 10000
N_PAD = 10240
N_EDGES = 320000
N_GRAPHS = 16
NW = 32
BLK = 128
NBLK = 79
E_PER_W = BLK * NBLK
E_PAD = E_PER_W * NW

_MESH = plsc.VectorSubcoreMesh(core_axis_name="c", subcore_axis_name="s")


def _zero_fill(ref1d, nwords):
    z = jnp.zeros((16,), jnp.float32)

    def body(i, c):
        ref1d[pl.ds(i * 16, 16)] = z
        return c

    lax.fori_loop(0, nwords // 16, body, None)


def _sc_deg_body(dst_hbm, out_hbm, idx_v, idx2_v, ones_v, buf_v, acc_sh,
                 sem, sem2):
    cid = lax.axis_index("c")
    sid = lax.axis_index("s")
    wid = cid * 16 + sid

    ov = jnp.ones((16,), jnp.float32)

    def fill(i, c):
        ones_v[pl.ds(i * 16, 16)] = ov
        return c

    lax.fori_loop(0, BLK // 16, fill, None)
    _zero_fill(buf_v, 640)
    pltpu.sync_copy(buf_v, acc_sh.at[pl.ds(sid * 640, 640)])
    plsc.subcore_barrier()

    base0 = wid * E_PER_W

    pltpu.sync_copy(dst_hbm.at[pl.ds(base0, BLK)], idx_v)
    pltpu.make_async_copy(ones_v, acc_sh.at[idx_v], sem).start(add=True)

    def blk(j, c):
        b = base0 + (2 * j + 1) * BLK
        pltpu.sync_copy(dst_hbm.at[pl.ds(b, BLK)], idx2_v)
        pltpu.make_async_copy(ones_v, acc_sh.at[idx2_v], sem2).start(add=True)
        pltpu.make_async_copy(ones_v, acc_sh.at[idx_v], sem).wait()
        pltpu.sync_copy(dst_hbm.at[pl.ds(b + BLK, BLK)], idx_v)
        pltpu.make_async_copy(ones_v, acc_sh.at[idx_v], sem).start(add=True)
        pltpu.make_async_copy(ones_v, acc_sh.at[idx2_v], sem2).wait()
        return c

    lax.fori_loop(0, (NBLK - 1) // 2, blk, None)
    pltpu.make_async_copy(ones_v, acc_sh.at[idx_v], sem).wait()
    plsc.subcore_barrier()

    pltpu.sync_copy(acc_sh.at[pl.ds(sid * 640, 640)], buf_v)
    pltpu.sync_copy(buf_v, out_hbm.at[cid, pl.ds(sid * 640, 640)])


_sc_deg = pl.kernel(
    _sc_deg_body,
    out_type=jax.ShapeDtypeStruct((2, N_PAD), jnp.float32),
    mesh=_MESH,
    scratch_types=[
        pltpu.VMEM((BLK,), jnp.int32),
        pltpu.VMEM((BLK,), jnp.int32),
        pltpu.VMEM((BLK,), jnp.float32),
        pltpu.VMEM((640,), jnp.float32),
        pltpu.VMEM_SHARED((N_PAD,), jnp.float32),
        pltpu.SemaphoreType.DMA,
        pltpu.SemaphoreType.DMA,
    ],
)


def _sc_prop_body(tab_hbm, src_hbm, dst_hbm, out_hbm,
                  sidx_v, didx_v, rows_v, sidx2_v, didx2_v, rows2_v,
                  buf_v, acc_sh, sem, sem2):
    cid = lax.axis_index("c")
    sid = lax.axis_index("s")
    wid = cid * 16 + sid

    z = jnp.zeros((16,), jnp.float32)

    def zfill(i, c):
        buf_v[i // 8, pl.ds((i % 8) * 16, 16)] = z
        return c

    lax.fori_loop(0, 32 * 8, zfill, None)

    def zblk(j, c):
        pltpu.sync_copy(buf_v, acc_sh.at[pl.ds(sid * 640 + j * 32, 32)])
        return c

    lax.fori_loop(0, 20, zblk, None)
    plsc.subcore_barrier()

    base0 = wid * E_PER_W

    def load_idx(b, si, di):
        pltpu.sync_copy(src_hbm.at[pl.ds(base0 + b * BLK, BLK)], si)
        pltpu.sync_copy(dst_hbm.at[pl.ds(base0 + b * BLK, BLK)], di)

    load_idx(0, sidx_v, didx_v)
    pltpu.make_async_copy(tab_hbm.at[sidx_v], rows_v, sem).start()

    def blk(j, c):
        load_idx(2 * j + 1, sidx2_v, didx2_v)
        pltpu.make_async_copy(tab_hbm.at[sidx2_v], rows2_v, sem2).start()
        pltpu.make_async_copy(tab_hbm.at[sidx_v], rows_v, sem).wait()
        pltpu.sync_copy(rows_v, acc_sh.at[didx_v], add=True)
        load_idx(2 * j + 2, sidx_v, didx_v)
        pltpu.make_async_copy(tab_hbm.at[sidx_v], rows_v, sem).start()
        pltpu.make_async_copy(tab_hbm.at[sidx2_v], rows2_v, sem2).wait()
        pltpu.sync_copy(rows2_v, acc_sh.at[didx2_v], add=True)
        return c

    lax.fori_loop(0, (NBLK - 1) // 2, blk, None)
    pltpu.make_async_copy(tab_hbm.at[sidx_v], rows_v, sem).wait()
    pltpu.sync_copy(rows_v, acc_sh.at[didx_v], add=True)
    plsc.subcore_barrier()

    def dblk(j, c):
        r = sid * 640 + j * 32
        pltpu.sync_copy(acc_sh.at[pl.ds(r, 32)], buf_v)
        pltpu.sync_copy(buf_v, out_hbm.at[cid, pl.ds(r, 32)])
        return c

    lax.fori_loop(0, 20, dblk, None)


_sc_prop = pl.kernel(
    _sc_prop_body,
    out_type=jax.ShapeDtypeStruct((2, N_PAD, 128), jnp.float32),
    mesh=_MESH,
    scratch_types=[
        pltpu.VMEM((BLK,), jnp.int32),
        pltpu.VMEM((BLK,), jnp.int32),
        pltpu.VMEM((BLK, 128), jnp.float32),
        pltpu.VMEM((BLK,), jnp.int32),
        pltpu.VMEM((BLK,), jnp.int32),
        pltpu.VMEM((BLK, 128), jnp.float32),
        pltpu.VMEM((32, 128), jnp.float32),
        pltpu.VMEM_SHARED((N_PAD, 128), jnp.float32),
        pltpu.SemaphoreType.DMA,
        pltpu.SemaphoreType.DMA,
    ],
)


def _tc_pre_body(x_ref, degt_ref, g0_ref, dinv_ref):
    deg = degt_ref[:, 0:1] + degt_ref[:, 1:2] + 1.0
    dinv = lax.rsqrt(deg)
    dinv_ref[...] = dinv
    g0_ref[...] = x_ref[...] * dinv


def _tc_mid_body(t0a_ref, t0b_ref, g0_ref, dinv_ref, w1_ref, b1_ref,
                 g1a_ref, g1b_ref):
    dinv = dinv_ref[...]
    ph0 = dinv * (t0a_ref[...] + t0b_ref[...] + g0_ref[...])
    h1 = jnp.maximum(
        jnp.dot(ph0, w1_ref[...], preferred_element_type=jnp.float32)
        + b1_ref[...], 0.0)
    g1 = dinv * h1
    g1a_ref[...] = g1[:, :128]
    g1b_ref[...] = g1[:, 128:]


def _tc_final_body(t1a0_ref, t1a1_ref, t1b0_ref, t1b1_ref, g1a_ref, g1b_ref,
                   dinv_ref, w2_ref, b2_ref, batchb_ref, wfc_ref, bfc_ref,
                   out_ref):
    dinv = dinv_ref[...]
    pha = dinv * (t1a0_ref[...] + t1a1_ref[...] + g1a_ref[...])
    phb = dinv * (t1b0_ref[...] + t1b1_ref[...] + g1b_ref[...])
    ph1 = jnp.concatenate([pha, phb], axis=1)
    h2 = jnp.maximum(
        jnp.dot(ph1, w2_ref[...], preferred_element_type=jnp.float32)
        + b2_ref[...], 0.0)
    gid = lax.broadcasted_iota(jnp.int32, (N_GRAPHS, N_NODES), 0)
    mask = (gid == batchb_ref[...]).astype(jnp.float32)
    pooled = jnp.dot(mask, h2, preferred_element_type=jnp.float32)
    counts = jnp.sum(mask, axis=1, keepdims=True)
    pooled = pooled / jnp.maximum(counts, 1.0)
    out_ref[...] = (
        jnp.dot(pooled, wfc_ref[...], preferred_element_type=jnp.float32)
        + bfc_ref[...])


def kernel(x, edge_index, batch, W1, b1, W2, b2, Wfc, bfc):
    npad = E_PAD - N_EDGES
    src = jnp.concatenate([edge_index[0], jnp.zeros((npad,), jnp.int32)])
    dst = jnp.concatenate(
        [edge_index[1], jnp.full((npad,), N_PAD - 1, jnp.int32)])

    degp = _sc_deg(dst)
    degt = jnp.transpose(degp[:, :N_NODES])

    g0, dinv = pl.pallas_call(
        _tc_pre_body,
        out_shape=(
            jax.ShapeDtypeStruct((N_NODES, 128), jnp.float32),
            jax.ShapeDtypeStruct((N_NODES, 1), jnp.float32),
        ),
    )(x, degt)

    t0 = _sc_prop(g0, src, dst)

    g1a, g1b = pl.pallas_call(
        _tc_mid_body,
        out_shape=(
            jax.ShapeDtypeStruct((N_NODES, 128), jnp.float32),
            jax.ShapeDtypeStruct((N_NODES, 128), jnp.float32),
        ),
    )(t0[0, :N_NODES], t0[1, :N_NODES], g0, dinv, W1,
      b1.reshape(1, 256))

    t1a = _sc_prop(g1a, src, dst)
    t1b = _sc_prop(g1b, src, dst)

    batchb = jnp.broadcast_to(batch[None, :], (N_GRAPHS, N_NODES))
    out = pl.pallas_call(
        _tc_final_body,
        out_shape=jax.ShapeDtypeStruct((N_GRAPHS, 1), jnp.float32),
    )(t1a[0, :N_NODES], t1a[1, :N_NODES], t1b[0, :N_NODES], t1b[1, :N_NODES],
      g1a, g1b, dinv, W2, b2.reshape(1, 256), batchb, Wfc,
      bfc.reshape(1, 1))
    return out[:, 0]


# speedup vs baseline: 1.8352x; 1.8352x over previous
import functools

import jax
import jax.numpy as jnp
from jax import lax
from jax.experimental import pallas as pl
from jax.experimental.pallas import tpu as pltpu
from jax.experimental.pallas import tpu_sc as plsc

N_NODES = 10000
N_PAD = 10240
N_EDGES = 320000
N_GRAPHS = 16
NW = 32
BLK = 128
NBLK = 79
E_PER_W = BLK * NBLK
E_PAD = E_PER_W * NW

_MESH = plsc.VectorSubcoreMesh(core_axis_name="c", subcore_axis_name="s")


def _zero_fill(ref1d, nwords):
    z = jnp.zeros((16,), jnp.float32)

    def body(i, c):
        ref1d[pl.ds(i * 16, 16)] = z
        return c

    lax.fori_loop(0, nwords // 16, body, None)


def _sc_deg_body(dst_hbm, out_hbm, idx_v, idx2_v, ones_v, buf_v, acc_sh,
                 sem, sem2):
    cid = lax.axis_index("c")
    sid = lax.axis_index("s")
    wid = cid * 16 + sid

    ov = jnp.ones((16,), jnp.float32)

    def fill(i, c):
        ones_v[pl.ds(i * 16, 16)] = ov
        return c

    lax.fori_loop(0, BLK // 16, fill, None)
    _zero_fill(buf_v, 640)
    pltpu.sync_copy(buf_v, acc_sh.at[pl.ds(sid * 640, 640)])
    plsc.subcore_barrier()

    base0 = wid * E_PER_W

    pltpu.sync_copy(dst_hbm.at[pl.ds(base0, BLK)], idx_v)
    pltpu.make_async_copy(ones_v, acc_sh.at[idx_v], sem).start(add=True)

    def blk(j, c):
        b = base0 + (2 * j + 1) * BLK
        pltpu.sync_copy(dst_hbm.at[pl.ds(b, BLK)], idx2_v)
        pltpu.make_async_copy(ones_v, acc_sh.at[idx2_v], sem2).start(add=True)
        pltpu.make_async_copy(ones_v, acc_sh.at[idx_v], sem).wait()
        pltpu.sync_copy(dst_hbm.at[pl.ds(b + BLK, BLK)], idx_v)
        pltpu.make_async_copy(ones_v, acc_sh.at[idx_v], sem).start(add=True)
        pltpu.make_async_copy(ones_v, acc_sh.at[idx2_v], sem2).wait()
        return c

    lax.fori_loop(0, (NBLK - 1) // 2, blk, None)
    pltpu.make_async_copy(ones_v, acc_sh.at[idx_v], sem).wait()
    plsc.subcore_barrier()

    pltpu.sync_copy(acc_sh.at[pl.ds(sid * 640, 640)], buf_v)
    pltpu.sync_copy(buf_v, out_hbm.at[cid, pl.ds(sid * 640, 640)])


_sc_deg = pl.kernel(
    _sc_deg_body,
    out_type=jax.ShapeDtypeStruct((2, N_PAD), jnp.float32),
    mesh=_MESH,
    scratch_types=[
        pltpu.VMEM((BLK,), jnp.int32),
        pltpu.VMEM((BLK,), jnp.int32),
        pltpu.VMEM((BLK,), jnp.float32),
        pltpu.VMEM((640,), jnp.float32),
        pltpu.VMEM_SHARED((N_PAD,), jnp.float32),
        pltpu.SemaphoreType.DMA,
        pltpu.SemaphoreType.DMA,
    ],
)


def _sc_prop_body(tab_hbm, src_hbm, dst_hbm, out_hbm,
                  sidx_v, didx_v, rows_v, sidx2_v, didx2_v, rows2_v,
                  buf_v, acc_sh, sem, sem2):
    cid = lax.axis_index("c")
    sid = lax.axis_index("s")
    wid = cid * 16 + sid

    z = jnp.zeros((16,), jnp.float32)

    def zfill(i, c):
        buf_v[i // 8, pl.ds((i % 8) * 16, 16)] = z
        return c

    lax.fori_loop(0, 32 * 8, zfill, None)

    def zblk(j, c):
        pltpu.sync_copy(buf_v, acc_sh.at[pl.ds(sid * 640 + j * 32, 32)])
        return c

    lax.fori_loop(0, 20, zblk, None)
    plsc.subcore_barrier()

    base0 = wid * E_PER_W

    def load_idx(b, si, di):
        pltpu.sync_copy(src_hbm.at[pl.ds(base0 + b * BLK, BLK)], si)
        pltpu.sync_copy(dst_hbm.at[pl.ds(base0 + b * BLK, BLK)], di)

    load_idx(0, sidx_v, didx_v)
    pltpu.make_async_copy(tab_hbm.at[sidx_v], rows_v, sem).start()

    def blk(j, c):
        load_idx(2 * j + 1, sidx2_v, didx2_v)
        pltpu.make_async_copy(tab_hbm.at[sidx2_v], rows2_v, sem2).start()
        pltpu.make_async_copy(tab_hbm.at[sidx_v], rows_v, sem).wait()
        pltpu.sync_copy(rows_v, acc_sh.at[didx_v], add=True)
        load_idx(2 * j + 2, sidx_v, didx_v)
        pltpu.make_async_copy(tab_hbm.at[sidx_v], rows_v, sem).start()
        pltpu.make_async_copy(tab_hbm.at[sidx2_v], rows2_v, sem2).wait()
        pltpu.sync_copy(rows2_v, acc_sh.at[didx2_v], add=True)
        return c

    lax.fori_loop(0, (NBLK - 1) // 2, blk, None)
    pltpu.make_async_copy(tab_hbm.at[sidx_v], rows_v, sem).wait()
    pltpu.sync_copy(rows_v, acc_sh.at[didx_v], add=True)
    plsc.subcore_barrier()

    def dblk(j, c):
        r = sid * 640 + j * 32
        pltpu.sync_copy(acc_sh.at[pl.ds(r, 32)], buf_v)
        pltpu.sync_copy(buf_v, out_hbm.at[cid, pl.ds(r, 32)])
        return c

    lax.fori_loop(0, 20, dblk, None)


_sc_prop = pl.kernel(
    _sc_prop_body,
    out_type=jax.ShapeDtypeStruct((2, N_PAD, 128), jnp.float32),
    mesh=_MESH,
    scratch_types=[
        pltpu.VMEM((BLK,), jnp.int32),
        pltpu.VMEM((BLK,), jnp.int32),
        pltpu.VMEM((BLK, 128), jnp.float32),
        pltpu.VMEM((BLK,), jnp.int32),
        pltpu.VMEM((BLK,), jnp.int32),
        pltpu.VMEM((BLK, 128), jnp.float32),
        pltpu.VMEM((32, 128), jnp.float32),
        pltpu.VMEM_SHARED((N_PAD, 128), jnp.float32),
        pltpu.SemaphoreType.DMA,
        pltpu.SemaphoreType.DMA,
    ],
)


def _tc_pre_body(x_ref, degt_ref, g0_ref, dinv_ref):
    deg = degt_ref[:, 0:1] + degt_ref[:, 1:2] + 1.0
    dinv = lax.rsqrt(deg)
    dinv_ref[...] = dinv
    g0_ref[...] = x_ref[...] * dinv


def _tc_mid_body(t0a_ref, t0b_ref, g0_ref, dinv_ref, w1_ref, b1_ref,
                 g1a_ref, g1b_ref):
    dinv = dinv_ref[...]
    ph0 = dinv * (t0a_ref[...] + t0b_ref[...] + g0_ref[...])
    h1 = jnp.maximum(
        jnp.dot(ph0, w1_ref[...], preferred_element_type=jnp.float32)
        + b1_ref[...], 0.0)
    g1 = dinv * h1
    g1a_ref[...] = g1[:, :128]
    g1b_ref[...] = g1[:, 128:]


def _tc_final_body(t1a0_ref, t1a1_ref, t1b0_ref, t1b1_ref, g1a_ref, g1b_ref,
                   dinv_ref, w2_ref, b2_ref, batchb_ref, wfc_ref, bfc_ref,
                   out_ref):
    dinv = dinv_ref[...]
    pha = dinv * (t1a0_ref[...] + t1a1_ref[...] + g1a_ref[...])
    phb = dinv * (t1b0_ref[...] + t1b1_ref[...] + g1b_ref[...])
    ph1 = jnp.concatenate([pha, phb], axis=1)
    h2 = jnp.maximum(
        jnp.dot(ph1, w2_ref[...], preferred_element_type=jnp.float32)
        + b2_ref[...], 0.0)
    gid = lax.broadcasted_iota(jnp.int32, (N_GRAPHS, N_NODES), 0)
    mask = (gid == batchb_ref[...]).astype(jnp.float32)
    pooled = jnp.dot(mask, h2, preferred_element_type=jnp.float32)
    counts = jnp.sum(mask, axis=1, keepdims=True)
    pooled = pooled / jnp.maximum(counts, 1.0)
    out_ref[...] = (
        jnp.dot(pooled, wfc_ref[...], preferred_element_type=jnp.float32)
        + bfc_ref[...])


def kernel(x, edge_index, batch, W1, b1, W2, b2, Wfc, bfc):
    npad = E_PAD - N_EDGES
    spread = jnp.arange(npad, dtype=jnp.int32)
    src = jnp.concatenate([edge_index[0], spread % N_NODES])
    dst = jnp.concatenate(
        [edge_index[1], N_NODES + (spread % (N_PAD - N_NODES))])

    degp = _sc_deg(dst)
    degt = jnp.transpose(degp[:, :N_NODES])

    g0, dinv = pl.pallas_call(
        _tc_pre_body,
        out_shape=(
            jax.ShapeDtypeStruct((N_NODES, 128), jnp.float32),
            jax.ShapeDtypeStruct((N_NODES, 1), jnp.float32),
        ),
    )(x, degt)

    t0 = _sc_prop(g0, src, dst)

    g1a, g1b = pl.pallas_call(
        _tc_mid_body,
        out_shape=(
            jax.ShapeDtypeStruct((N_NODES, 128), jnp.float32),
            jax.ShapeDtypeStruct((N_NODES, 128), jnp.float32),
        ),
    )(t0[0, :N_NODES], t0[1, :N_NODES], g0, dinv, W1,
      b1.reshape(1, 256))

    t1a = _sc_prop(g1a, src, dst)
    t1b = _sc_prop(g1b, src, dst)

    batchb = jnp.broadcast_to(batch[None, :], (N_GRAPHS, N_NODES))
    out = pl.pallas_call(
        _tc_final_body,
        out_shape=jax.ShapeDtypeStruct((N_GRAPHS, 1), jnp.float32),
    )(t1a[0, :N_NODES], t1a[1, :N_NODES], t1b[0, :N_NODES], t1b[1, :N_NODES],
      g1a, g1b, dinv, W2, b2.reshape(1, 256), batchb, Wfc,
      bfc.reshape(1, 1))
    return out[:, 0]


# speedup vs baseline: 1.9121x; 1.0419x over previous
import functools

import jax
import jax.numpy as jnp
from jax import lax
from jax.experimental import pallas as pl
from jax.experimental.pallas import tpu as pltpu
from jax.experimental.pallas import tpu_sc as plsc

N_NODES = 10000
N_PAD = 10240
N_EDGES = 320000
N_GRAPHS = 16
NW = 32
BLK = 128
NBLK = 79
E_PER_W = BLK * NBLK
E_PAD = E_PER_W * NW

_MESH = plsc.VectorSubcoreMesh(core_axis_name="c", subcore_axis_name="s")


def _zero_fill(ref1d, nwords):
    z = jnp.zeros((16,), jnp.float32)

    def body(i, c):
        ref1d[pl.ds(i * 16, 16)] = z
        return c

    lax.fori_loop(0, nwords // 16, body, None)


def _sc_deg_body(dst_hbm, out_hbm, idx_v, idx2_v, ones_v, buf_v, acc_sh,
                 sem, sem2):
    cid = lax.axis_index("c")
    sid = lax.axis_index("s")
    wid = cid * 16 + sid

    ov = jnp.ones((16,), jnp.float32)

    def fill(i, c):
        ones_v[pl.ds(i * 16, 16)] = ov
        return c

    lax.fori_loop(0, BLK // 16, fill, None)
    _zero_fill(buf_v, 640)
    pltpu.sync_copy(buf_v, acc_sh.at[pl.ds(sid * 640, 640)])
    plsc.subcore_barrier()

    base0 = wid * E_PER_W

    pltpu.sync_copy(dst_hbm.at[pl.ds(base0, BLK)], idx_v)
    pltpu.make_async_copy(ones_v, acc_sh.at[idx_v], sem).start(add=True)

    def blk(j, c):
        b = base0 + (2 * j + 1) * BLK
        pltpu.sync_copy(dst_hbm.at[pl.ds(b, BLK)], idx2_v)
        pltpu.make_async_copy(ones_v, acc_sh.at[idx2_v], sem2).start(add=True)
        pltpu.make_async_copy(ones_v, acc_sh.at[idx_v], sem).wait()
        pltpu.sync_copy(dst_hbm.at[pl.ds(b + BLK, BLK)], idx_v)
        pltpu.make_async_copy(ones_v, acc_sh.at[idx_v], sem).start(add=True)
        pltpu.make_async_copy(ones_v, acc_sh.at[idx2_v], sem2).wait()
        return c

    lax.fori_loop(0, (NBLK - 1) // 2, blk, None)
    pltpu.make_async_copy(ones_v, acc_sh.at[idx_v], sem).wait()
    plsc.subcore_barrier()

    pltpu.sync_copy(acc_sh.at[pl.ds(sid * 640, 640)], buf_v)
    pltpu.sync_copy(buf_v, out_hbm.at[cid, pl.ds(sid * 640, 640)])


_sc_deg = pl.kernel(
    _sc_deg_body,
    out_type=jax.ShapeDtypeStruct((2, N_PAD), jnp.float32),
    mesh=_MESH,
    scratch_types=[
        pltpu.VMEM((BLK,), jnp.int32),
        pltpu.VMEM((BLK,), jnp.int32),
        pltpu.VMEM((BLK,), jnp.float32),
        pltpu.VMEM((640,), jnp.float32),
        pltpu.VMEM_SHARED((N_PAD,), jnp.float32),
        pltpu.SemaphoreType.DMA,
        pltpu.SemaphoreType.DMA,
    ],
)


def _zero_acc_slice(buf_v, acc_sh, sid):
    z = jnp.zeros((16,), jnp.float32)

    def zfill(i, c):
        buf_v[i // 8, pl.ds((i % 8) * 16, 16)] = z
        return c

    lax.fori_loop(0, 32 * 8, zfill, None)

    def zblk(j, c):
        pltpu.sync_copy(buf_v, acc_sh.at[pl.ds(sid * 640 + j * 32, 32)])
        return c

    lax.fori_loop(0, 20, zblk, None)


def _dump_acc_slice(buf_v, acc_sh, out_hbm, cid, sid):
    def dblk(j, c):
        r = sid * 640 + j * 32
        pltpu.sync_copy(acc_sh.at[pl.ds(r, 32)], buf_v)
        pltpu.sync_copy(buf_v, out_hbm.at[cid, pl.ds(r, 32)])
        return c

    lax.fori_loop(0, 20, dblk, None)


def _edge_ring(tab_hbm, src_hbm, dst_hbm, acc_sh, base0, nblk,
               sidx_v, didx_v, rows_v, sidx2_v, didx2_v, rows2_v, sem, sem2):
    def load_idx(b, si, di):
        pltpu.sync_copy(src_hbm.at[pl.ds(base0 + b * BLK, BLK)], si)
        pltpu.sync_copy(dst_hbm.at[pl.ds(base0 + b * BLK, BLK)], di)

    load_idx(0, sidx_v, didx_v)
    pltpu.make_async_copy(tab_hbm.at[sidx_v], rows_v, sem).start()

    def blk(j, c):
        load_idx(2 * j + 1, sidx2_v, didx2_v)
        pltpu.make_async_copy(tab_hbm.at[sidx2_v], rows2_v, sem2).start()
        pltpu.make_async_copy(tab_hbm.at[sidx_v], rows_v, sem).wait()
        pltpu.sync_copy(rows_v, acc_sh.at[didx_v], add=True)
        load_idx(2 * j + 2, sidx_v, didx_v)
        pltpu.make_async_copy(tab_hbm.at[sidx_v], rows_v, sem).start()
        pltpu.make_async_copy(tab_hbm.at[sidx2_v], rows2_v, sem2).wait()
        pltpu.sync_copy(rows2_v, acc_sh.at[didx2_v], add=True)
        return c

    npairs = (nblk - 1) // 2
    lax.fori_loop(0, npairs, blk, None)
    pltpu.make_async_copy(tab_hbm.at[sidx_v], rows_v, sem).wait()
    pltpu.sync_copy(rows_v, acc_sh.at[didx_v], add=True)
    if nblk - (2 * npairs + 1) == 1:
        load_idx(nblk - 1, sidx2_v, didx2_v)
        pltpu.make_async_copy(tab_hbm.at[sidx2_v], rows2_v, sem2).start()
        pltpu.make_async_copy(tab_hbm.at[sidx2_v], rows2_v, sem2).wait()
        pltpu.sync_copy(rows2_v, acc_sh.at[didx2_v], add=True)


def _sc_prop_body(tab_hbm, src_hbm, dst_hbm, out_hbm,
                  sidx_v, didx_v, rows_v, sidx2_v, didx2_v, rows2_v,
                  buf_v, acc_sh, sem, sem2):
    cid = lax.axis_index("c")
    sid = lax.axis_index("s")
    wid = cid * 16 + sid

    _zero_acc_slice(buf_v, acc_sh, sid)
    plsc.subcore_barrier()
    _edge_ring(tab_hbm, src_hbm, dst_hbm, acc_sh, wid * E_PER_W, NBLK,
               sidx_v, didx_v, rows_v, sidx2_v, didx2_v, rows2_v, sem, sem2)
    plsc.subcore_barrier()
    _dump_acc_slice(buf_v, acc_sh, out_hbm, cid, sid)


_PROP_SCRATCH = [
    pltpu.VMEM((BLK,), jnp.int32),
    pltpu.VMEM((BLK,), jnp.int32),
    pltpu.VMEM((BLK, 128), jnp.float32),
    pltpu.VMEM((BLK,), jnp.int32),
    pltpu.VMEM((BLK,), jnp.int32),
    pltpu.VMEM((BLK, 128), jnp.float32),
    pltpu.VMEM((32, 128), jnp.float32),
    pltpu.VMEM_SHARED((N_PAD, 128), jnp.float32),
    pltpu.SemaphoreType.DMA,
    pltpu.SemaphoreType.DMA,
]

_sc_prop = pl.kernel(
    _sc_prop_body,
    out_type=jax.ShapeDtypeStruct((2, N_PAD, 128), jnp.float32),
    mesh=_MESH,
    scratch_types=list(_PROP_SCRATCH),
)

E_PER_T2 = E_PAD // 16
NBLK2 = E_PER_T2 // BLK


def _sc_prop2_body(ga_hbm, gb_hbm, src_hbm, dst_hbm, out_hbm,
                   sidx_v, didx_v, rows_v, sidx2_v, didx2_v, rows2_v,
                   buf_v, acc_sh, sem, sem2):
    cid = lax.axis_index("c")
    sid = lax.axis_index("s")

    _zero_acc_slice(buf_v, acc_sh, sid)
    plsc.subcore_barrier()
    base0 = sid * E_PER_T2

    @pl.when(cid == 0)
    def _():
        _edge_ring(ga_hbm, src_hbm, dst_hbm, acc_sh, base0, NBLK2,
                   sidx_v, didx_v, rows_v, sidx2_v, didx2_v, rows2_v,
                   sem, sem2)

    @pl.when(cid == 1)
    def _():
        _edge_ring(gb_hbm, src_hbm, dst_hbm, acc_sh, base0, NBLK2,
                   sidx_v, didx_v, rows_v, sidx2_v, didx2_v, rows2_v,
                   sem, sem2)

    plsc.subcore_barrier()
    _dump_acc_slice(buf_v, acc_sh, out_hbm, cid, sid)


_sc_prop2 = pl.kernel(
    _sc_prop2_body,
    out_type=jax.ShapeDtypeStruct((2, N_PAD, 128), jnp.float32),
    mesh=_MESH,
    scratch_types=list(_PROP_SCRATCH),
)


def _tc_pre_body(x_ref, degt_ref, g0_ref, dinv_ref):
    deg = degt_ref[:, 0:1] + degt_ref[:, 1:2] + 1.0
    dinv = lax.rsqrt(deg)
    dinv_ref[...] = dinv
    g0_ref[...] = x_ref[...] * dinv


def _tc_mid_body(t0a_ref, t0b_ref, g0_ref, dinv_ref, w1_ref, b1_ref,
                 g1a_ref, g1b_ref):
    dinv = dinv_ref[...]
    ph0 = dinv * (t0a_ref[...] + t0b_ref[...] + g0_ref[...])
    h1 = jnp.maximum(
        jnp.dot(ph0, w1_ref[...], preferred_element_type=jnp.float32)
        + b1_ref[...], 0.0)
    g1 = dinv * h1
    g1a_ref[...] = g1[:, :128]
    g1b_ref[...] = g1[:, 128:]


def _tc_final_body(t1a_ref, t1b_ref, g1a_ref, g1b_ref,
                   dinv_ref, w2_ref, b2_ref, batchb_ref, wfc_ref, bfc_ref,
                   out_ref):
    dinv = dinv_ref[...]
    pha = dinv * (t1a_ref[...] + g1a_ref[...])
    phb = dinv * (t1b_ref[...] + g1b_ref[...])
    ph1 = jnp.concatenate([pha, phb], axis=1)
    h2 = jnp.maximum(
        jnp.dot(ph1, w2_ref[...], preferred_element_type=jnp.float32)
        + b2_ref[...], 0.0)
    gid = lax.broadcasted_iota(jnp.int32, (N_GRAPHS, N_NODES), 0)
    mask = (gid == batchb_ref[...]).astype(jnp.float32)
    pooled = jnp.dot(mask, h2, preferred_element_type=jnp.float32)
    counts = jnp.sum(mask, axis=1, keepdims=True)
    pooled = pooled / jnp.maximum(counts, 1.0)
    out_ref[...] = (
        jnp.dot(pooled, wfc_ref[...], preferred_element_type=jnp.float32)
        + bfc_ref[...])


def kernel(x, edge_index, batch, W1, b1, W2, b2, Wfc, bfc):
    npad = E_PAD - N_EDGES
    spread = jnp.arange(npad, dtype=jnp.int32)
    src = jnp.concatenate([edge_index[0], spread % N_NODES])
    dst = jnp.concatenate(
        [edge_index[1], N_NODES + (spread % (N_PAD - N_NODES))])

    degp = _sc_deg(dst)
    degt = jnp.transpose(degp[:, :N_NODES])

    g0, dinv = pl.pallas_call(
        _tc_pre_body,
        out_shape=(
            jax.ShapeDtypeStruct((N_NODES, 128), jnp.float32),
            jax.ShapeDtypeStruct((N_NODES, 1), jnp.float32),
        ),
    )(x, degt)

    t0 = _sc_prop(g0, src, dst)

    g1a, g1b = pl.pallas_call(
        _tc_mid_body,
        out_shape=(
            jax.ShapeDtypeStruct((N_NODES, 128), jnp.float32),
            jax.ShapeDtypeStruct((N_NODES, 128), jnp.float32),
        ),
    )(t0[0, :N_NODES], t0[1, :N_NODES], g0, dinv, W1,
      b1.reshape(1, 256))

    t1 = _sc_prop2(g1a, g1b, src, dst)

    batchb = jnp.broadcast_to(batch[None, :], (N_GRAPHS, N_NODES))
    out = pl.pallas_call(
        _tc_final_body,
        out_shape=jax.ShapeDtypeStruct((N_GRAPHS, 1), jnp.float32),
    )(t1[0, :N_NODES], t1[1, :N_NODES],
      g1a, g1b, dinv, W2, b2.reshape(1, 256), batchb, Wfc,
      bfc.reshape(1, 1))
    return out[:, 0]
